# baseline (speedup 1.0000x reference)
import functools

import jax
import jax.numpy as jnp
from jax import lax
from jax.experimental import pallas as pl
from jax.experimental.pallas import tpu as pltpu
from jax.experimental.pallas import tpu_sc as plsc

DIM = 64
NC = 2
NS = 16
NW = NC * NS
BPW = 128
NBUF = 4


@functools.lru_cache(maxsize=None)
def _build(batch: int, seq: int):
    assert batch == NW * BPW and seq >= NBUF
    mesh = plsc.VectorSubcoreMesh(core_axis_name="c", subcore_axis_name="s")

    @functools.partial(
        pl.kernel,
        mesh=mesh,
        out_type=jax.ShapeDtypeStruct((seq, DIM // 8, NW, 8, BPW), jnp.float32),
        compiler_params=pltpu.CompilerParams(
            use_tc_tiling_on_sc=False, needs_layout_passes=False
        ),
        scratch_types=[
            pltpu.VMEM((BPW, seq), jnp.int32),
            pltpu.VMEM((seq, BPW), jnp.int32),
            pltpu.VMEM((NBUF, BPW, DIM), jnp.float32),
            pltpu.VMEM((2, DIM // 8, 8, BPW), jnp.float32),
            pltpu.SemaphoreType.DMA((NBUF,)),
            pltpu.SemaphoreType.DMA((2,)),
        ],
    )
    def gather_kernel(table_hbm, idx_hbm, out_hbm, idx_v, idx_t, rows_v,
                      out_v, gsem, wsem):
        wid = lax.axis_index("s") * NC + lax.axis_index("c")
        base = wid * BPW
        pltpu.sync_copy(idx_hbm.at[pl.ds(base, BPW)], idx_v)

        lane = lax.iota(jnp.int32, 16)
        lanes = [lane + g * 16 for g in range(BPW // 16)]

        @plsc.parallel_loop(0, seq)
        def tr_idx(s):
            for g in range(BPW // 16):
                v = plsc.load_gather(
                    idx_v, [lanes[g], jnp.full((16,), s, jnp.int32)]
                )
                idx_t[s, pl.ds(g * 16, 16)] = v

        for b in range(NBUF):
            pltpu.async_copy(table_hbm.at[idx_t.at[b]], rows_v.at[b],
                             gsem.at[b])

        def sentence(s, carry):
            gb = lax.rem(s, NBUF)
            ob = lax.rem(s, 2)
            rv = rows_v.at[gb]
            ov = out_v.at[ob]
            pltpu.make_async_copy(
                table_hbm.at[idx_t.at[s]], rv, gsem.at[gb]
            ).wait()

            @pl.when(s >= 2)
            def _():
                pltpu.make_async_copy(
                    ov, out_hbm.at[s].at[:, wid], wsem.at[ob]
                ).wait()

            @plsc.parallel_loop(0, DIM // 8, unroll=2)
            def tr_rows(fo):
                ovf = ov.at[fo]
                f0 = fo * 8
                for fr in range(8):
                    fvec = jnp.full((16,), f0 + fr, jnp.int32)
                    for g in range(BPW // 16):
                        v = plsc.load_gather(rv, [lanes[g], fvec])
                        ovf[fr, pl.ds(g * 16, 16)] = v

            pltpu.async_copy(ov, out_hbm.at[s].at[:, wid], wsem.at[ob])

            @pl.when(s + NBUF < seq)
            def _():
                pltpu.async_copy(
                    table_hbm.at[idx_t.at[s + NBUF]], rv, gsem.at[gb]
                )
            return carry

        lax.fori_loop(0, seq, sentence, 0)

        for k in range(2):
            s = seq - 2 + k
            pltpu.make_async_copy(
                out_v.at[s % 2], out_hbm.at[s].at[:, wid], wsem.at[s % 2]
            ).wait()

    return gather_kernel


def kernel(text, table):
    batch, seq = text.shape
    raw = _build(batch, seq)(table, text.astype(jnp.int32))
    return raw.transpose(2, 4, 0, 1, 3).reshape(batch, seq, DIM)


# speedup vs baseline: 1.1377x; 1.1377x over previous
import functools

import jax
import jax.numpy as jnp
from jax import lax
from jax.experimental import pallas as pl
from jax.experimental.pallas import tpu as pltpu
from jax.experimental.pallas import tpu_sc as plsc

DIM = 64
NC = 2
NS = 16
NW = NC * NS
SPC = 1
NBUF = 8
DIM_PAD = 128


@functools.lru_cache(maxsize=None)
def _build(n_sent: int, seq: int):
    seq_pad = (seq + 7) // 8 * 8
    spw = n_sent // NW
    cpw = spw // SPC
    assert cpw % NBUF == 0 and cpw // NBUF >= 2
    n_main_groups = cpw // NBUF - 1
    mesh = plsc.VectorSubcoreMesh(core_axis_name="c", subcore_axis_name="s")

    @functools.partial(
        pl.kernel,
        mesh=mesh,
        out_type=jax.ShapeDtypeStruct((n_sent, seq_pad, DIM_PAD), jnp.float32),
        compiler_params=pltpu.CompilerParams(use_tc_tiling_on_sc=False),
        scratch_types=[
            pltpu.VMEM((spw, seq), jnp.int32),
            pltpu.VMEM((NBUF, seq, DIM), jnp.float32),
            pltpu.SemaphoreType.DMA((NBUF,)),
            pltpu.SemaphoreType.DMA((NBUF,)),
        ],
    )
    def gather_kernel(table_hbm, idx_hbm, out_hbm, idx_v, rows_v, gsem, wsem):
        wid = lax.axis_index("s") * NC + lax.axis_index("c")
        base = wid * spw
        pltpu.sync_copy(idx_hbm.at[pl.ds(base, spw)], idx_v)

        def odst(j):
            return out_hbm.at[base + j].at[pl.ds(0, seq), pl.ds(0, DIM)]

        for b in range(NBUF):
            pltpu.async_copy(table_hbm.at[idx_v.at[b]], rows_v.at[b], gsem.at[b])

        def group(g, carry):
            j0 = g * NBUF
            for b in range(NBUF):
                j = j0 + b
                pltpu.make_async_copy(
                    table_hbm.at[idx_v.at[j]], rows_v.at[b], gsem.at[b]
                ).wait()
                w = pltpu.async_copy(rows_v.at[b], odst(j), wsem.at[b])
                w.wait()
                pltpu.async_copy(
                    table_hbm.at[idx_v.at[j + NBUF]], rows_v.at[b], gsem.at[b]
                )
            return carry

        lax.fori_loop(0, n_main_groups, group, 0)

        j0 = n_main_groups * NBUF
        for b in range(NBUF):
            j = j0 + b
            pltpu.make_async_copy(
                table_hbm.at[idx_v.at[j]], rows_v.at[b], gsem.at[b]
            ).wait()
            pltpu.async_copy(rows_v.at[b], odst(j), wsem.at[b])
        for b in range(NBUF):
            j = j0 + b
            pltpu.make_async_copy(
                rows_v.at[b], odst(j), wsem.at[b]
            ).wait()

    return gather_kernel


def kernel(text, table):
    n_sent, seq = text.shape
    raw = _build(n_sent, seq)(table, text.astype(jnp.int32))
    return lax.slice(raw, (0, 0, 0), (n_sent, seq, DIM))
